# baseline (speedup 1.0000x reference)
import functools

import jax
import jax.numpy as jnp
from jax import lax
from jax.experimental import pallas as pl
from jax.experimental.pallas import tpu as pltpu
from jax.experimental.pallas import tpu_sc as plsc

N = 10000
E = 320000
D = 128

NC = 2
NS = 16
NW = NC * NS
CHUNK = 96
Q = -(-E // (NW * CHUNK)) * CHUNK
EPAD = Q * NW - E
MAIN = Q // CHUNK
STRIPE = 632
LAST = N - STRIPE * (NS - 1)


NBUF = 2
NIDX = 3
ROUNDS = -(-MAIN // (NBUF * NIDX))


def _scale_rows(rows_ref, srows_ref, w_ref):
    for g in range(CHUNK // 16):
        w16 = w_ref[pl.ds(g * 16, 16)]
        for l in range(16):
            wb = w16[l]
            e = g * 16 + l
            for j in range(D // 16):
                sl = pl.ds(j * 16, 16)
                srows_ref[e, sl] = rows_ref[e, sl] * wb


def _spmm_sc_body(x_hbm, src_hbm, dst_hbm, w_hbm, zeros_hbm, out_hbm,
                  src_all, srows, w_b, dst_b, rows_b, gsem, isem, ssem,
                  acc_sh):
    c = lax.axis_index("c")
    s = lax.axis_index("s")
    wid = s * NC + c
    base = wid * Q

    @pl.when(s < NS - 1)
    def _():
        pltpu.sync_copy(zeros_hbm.at[pl.ds(s * STRIPE, STRIPE)],
                        acc_sh.at[pl.ds(s * STRIPE, STRIPE)])

    @pl.when(s == NS - 1)
    def _():
        pltpu.sync_copy(zeros_hbm.at[pl.ds((NS - 1) * STRIPE, LAST)],
                        acc_sh.at[pl.ds((NS - 1) * STRIPE, LAST)])

    pltpu.sync_copy(src_hbm.at[pl.ds(base, Q)], src_all)

    def src_idx(i):
        return src_all.at[pl.ds(i * CHUNK, CHUNK)]

    def start_slot(i, b, bi):
        pltpu.async_copy(dst_hbm.at[pl.ds(base + i * CHUNK, CHUNK)],
                         dst_b[bi].at[0], isem[bi])
        pltpu.async_copy(w_hbm.at[pl.ds(base + i * CHUNK, CHUNK)],
                         w_b[bi], isem[bi])
        pltpu.async_copy(x_hbm.at[src_idx(i)], rows_b[b], gsem[b])

    def scatter_wait(bi):
        pltpu.make_async_copy(srows, acc_sh.at[dst_b[bi].at[0]],
                              ssem).wait()

    start_slot(0, 0, 0)
    plsc.subcore_barrier()

    def round_body(r, _):
        for u in range(NBUF * NIDX):
            i = r * NBUF * NIDX + u
            b, bi = u % NBUF, u % NIDX
            bn, bin_ = (u + 1) % NBUF, (u + 1) % NIDX

            @pl.when(i + 1 < MAIN)
            def _():
                start_slot(i + 1, bn, bin_)

            @pl.when(i < MAIN)
            def _():
                pltpu.make_async_copy(
                    x_hbm.at[src_idx(i)], rows_b[b], gsem[b]).wait()
                pltpu.make_async_copy(
                    w_hbm.at[pl.ds(base + i * CHUNK, CHUNK)],
                    w_b[bi], isem[bi]).wait()
                pltpu.make_async_copy(
                    dst_hbm.at[pl.ds(base + i * CHUNK, CHUNK)],
                    dst_b[bi].at[0], isem[bi]).wait()

            @pl.when((i >= 1) & (i < MAIN))
            def _():
                scatter_wait((u - 1) % NIDX)

            @pl.when(i < MAIN)
            def _():
                _scale_rows(rows_b[b], srows, w_b[bi])
                pltpu.async_copy(srows, acc_sh.at[dst_b[bi].at[0]],
                                 ssem, add=True)
        return 0
    lax.fori_loop(0, ROUNDS, round_body, 0)

    scatter_wait((MAIN - 1) % NIDX)

    plsc.subcore_barrier()

    @pl.when(s < NS - 1)
    def _():
        pltpu.sync_copy(acc_sh.at[pl.ds(s * STRIPE, STRIPE)],
                        out_hbm.at[c, pl.ds(s * STRIPE, STRIPE)])

    @pl.when(s == NS - 1)
    def _():
        pltpu.sync_copy(acc_sh.at[pl.ds((NS - 1) * STRIPE, LAST)],
                        out_hbm.at[c, pl.ds((NS - 1) * STRIPE, LAST)])


@functools.lru_cache(maxsize=None)
def _spmm_sc():
    return pl.kernel(
        _spmm_sc_body,
        out_type=jax.ShapeDtypeStruct((NC, N, D), jnp.float32),
        mesh=plsc.VectorSubcoreMesh(core_axis_name="c", subcore_axis_name="s",
                                    num_cores=NC, num_subcores=NS),
        scratch_types=[
            pltpu.VMEM((Q,), jnp.int32),
            pltpu.VMEM((CHUNK, D), jnp.float32),
            [pltpu.VMEM((CHUNK,), jnp.float32) for _ in range(NIDX)],
            [pltpu.VMEM((1, CHUNK), jnp.int32) for _ in range(NIDX)],
            [pltpu.VMEM((CHUNK, D), jnp.float32) for _ in range(NBUF)],
            [pltpu.SemaphoreType.DMA for _ in range(NBUF)],
            [pltpu.SemaphoreType.DMA for _ in range(NIDX)],
            pltpu.SemaphoreType.DMA,
            pltpu.VMEM_SHARED((N, D), jnp.float32),
        ],
    )


def _pad_adj(src, dst, w):
    zi = jnp.zeros((EPAD,), jnp.int32)
    return (jnp.concatenate([src, zi]), jnp.concatenate([dst, zi]),
            jnp.concatenate([w, jnp.zeros((EPAD,), jnp.float32)]))


def _spmm(x, adj):
    src, dst, w = adj
    return _spmm_sc()(x, src, dst, w, jnp.zeros((N, D), jnp.float32))


BN = 2000


def _mm_body(x_ref, w_ref, b_ref, o_ref):
    o_ref[...] = jnp.dot(x_ref[...], w_ref[...],
                         preferred_element_type=jnp.float32) + b_ref[...]


def _mm_fused_body(p_ref, w_ref, b_ref, o_ref):
    x = jax.nn.relu(p_ref[0] + p_ref[1])
    o_ref[...] = jnp.dot(x, w_ref[...],
                         preferred_element_type=jnp.float32) + b_ref[...]


def _mm_fused_id_body(p_ref, h_ref, w_ref, b_ref, o_ref):
    x = jax.nn.relu(p_ref[0] + p_ref[1] + h_ref[...])
    o_ref[...] = jnp.dot(x, w_ref[...],
                         preferred_element_type=jnp.float32) + b_ref[...]


def _relu_sum_body(p_ref, o_ref):
    o_ref[...] = jax.nn.relu(p_ref[0] + p_ref[1])


def _norm_body(p_ref, o_ref):
    r = p_ref[0] + p_ref[1]
    n = jnp.sqrt(jnp.sum(r * r, axis=-1, keepdims=True))
    o_ref[...] = r / (n + 1e-9)


_row_spec = pl.BlockSpec((BN, D), lambda i: (i, 0))
_p_spec = pl.BlockSpec((2, BN, D), lambda i: (0, i, 0))
_w_spec = pl.BlockSpec((D, D), lambda i: (0, 0))
_b_spec = pl.BlockSpec((1, D), lambda i: (0, 0))
_out_nd = jax.ShapeDtypeStruct((N, D), jnp.float32)


def _mm(x, w, b):
    return pl.pallas_call(
        _mm_body, grid=(N // BN,),
        in_specs=[_row_spec, _w_spec, _b_spec],
        out_specs=_row_spec, out_shape=_out_nd)(x, w, b.reshape(1, D))


def _mm_fused(p, w, b):
    return pl.pallas_call(
        _mm_fused_body, grid=(N // BN,),
        in_specs=[_p_spec, _w_spec, _b_spec],
        out_specs=_row_spec, out_shape=_out_nd)(p, w, b.reshape(1, D))


def _mm_fused_id(p, h, w, b):
    return pl.pallas_call(
        _mm_fused_id_body, grid=(N // BN,),
        in_specs=[_p_spec, _row_spec, _w_spec, _b_spec],
        out_specs=_row_spec, out_shape=_out_nd)(p, h, w, b.reshape(1, D))


def _relu_sum(p):
    return pl.pallas_call(
        _relu_sum_body, grid=(N // BN,),
        in_specs=[_p_spec], out_specs=_row_spec, out_shape=_out_nd)(p)


def _norm(p):
    return pl.pallas_call(
        _norm_body, grid=(N // BN,),
        in_specs=[_p_spec], out_specs=_row_spec, out_shape=_out_nd)(p)


def _one_type(feat, W1, b1, W2, b2, src, dst, w, src0, dst0, w0, identity):
    adj = _pad_adj(src, dst, w)
    adj0 = _pad_adj(src0, dst0, w0)
    h = _mm(feat, W1, b1)
    p1 = _spmm(h, adj)
    if identity:
        h2in = _mm_fused_id(p1, h, W2, b2)
    else:
        h2in = _mm_fused(p1, W2, b2)
    p2 = _spmm(h2in, adj)
    x2 = _relu_sum(p2)
    p3 = _spmm(x2, adj0)
    return _norm(p3)


def kernel(feat_1, W1_1, b1_1, W2_1, b2_1, src_11, dst_11, w_11,
           src_01, dst_01, w_01,
           feat_2, W1_2, b1_2, W2_2, b2_2, src_22, dst_22, w_22,
           src_02, dst_02, w_02,
           feat_3, W1_3, b1_3, W2_3, b2_3, src_33, dst_33, w_33,
           src_03, dst_03, w_03, epoch):
    r1 = _one_type(feat_1, W1_1, b1_1, W2_1, b2_1, src_11, dst_11, w_11,
                   src_01, dst_01, w_01, identity=True)
    r2 = _one_type(feat_2, W1_2, b1_2, W2_2, b2_2, src_22, dst_22, w_22,
                   src_02, dst_02, w_02, identity=False)
    r3 = _one_type(feat_3, W1_3, b1_3, W2_3, b2_3, src_33, dst_33, w_33,
                   src_03, dst_03, w_03, identity=False)
    return jnp.stack([r1, r2, r3], axis=0)


# speedup vs baseline: 1.6095x; 1.6095x over previous
import functools

import jax
import jax.numpy as jnp
from jax import lax
from jax.experimental import pallas as pl
from jax.experimental.pallas import tpu as pltpu
from jax.experimental.pallas import tpu_sc as plsc

N = 10000
E = 320000
D = 128

NC = 2
NS = 16
NW = NC * NS
CHUNK = 80
Q = -(-E // (NW * CHUNK)) * CHUNK
EPAD = Q * NW - E
MAIN = Q // CHUNK
STRIPE = 632
LAST = N - STRIPE * (NS - 1)


NBUF = 2
NIDX = 4
UNROLL = 4
ROUNDS = -(-MAIN // UNROLL)


def _scale_rows(rows_ref, srows_ref, w_ref):
    for g in range(CHUNK // 16):
        w16 = w_ref[pl.ds(g * 16, 16)]
        for l in range(16):
            wb = w16[l]
            e = g * 16 + l
            for j in range(D // 16):
                sl = pl.ds(j * 16, 16)
                srows_ref[e, sl] = rows_ref[e, sl] * wb


def _spmm_sc_body(x_hbm, src_hbm, dst_hbm, w_hbm, zeros_hbm, out_hbm,
                  src_b, srows, w_b, dst_b, rows_b, gsem, isem, ssem,
                  acc_sh):
    c = lax.axis_index("c")
    s = lax.axis_index("s")
    wid = s * NC + c
    base = wid * Q

    @pl.when(s < NS - 1)
    def _():
        pltpu.sync_copy(zeros_hbm.at[pl.ds(s * STRIPE, STRIPE)],
                        acc_sh.at[pl.ds(s * STRIPE, STRIPE)])

    @pl.when(s == NS - 1)
    def _():
        pltpu.sync_copy(zeros_hbm.at[pl.ds((NS - 1) * STRIPE, LAST)],
                        acc_sh.at[pl.ds((NS - 1) * STRIPE, LAST)])

    def idx_start(i, r):
        sl = pl.ds(base + i * CHUNK, CHUNK)
        pltpu.async_copy(src_hbm.at[sl], src_b[r], isem[r])
        pltpu.async_copy(w_hbm.at[sl], w_b[r], isem[r])
        pltpu.async_copy(dst_hbm.at[sl], dst_b[r].at[0], isem[r])

    def idx_wait(i, r):
        sl = pl.ds(base + i * CHUNK, CHUNK)
        pltpu.make_async_copy(src_hbm.at[sl], src_b[r], isem[r]).wait()
        pltpu.make_async_copy(w_hbm.at[sl], w_b[r], isem[r]).wait()
        pltpu.make_async_copy(dst_hbm.at[sl], dst_b[r].at[0], isem[r]).wait()

    def gather_start(b, r):
        pltpu.async_copy(x_hbm.at[src_b[r]], rows_b[b], gsem[b])

    def gather_wait(b, r):
        pltpu.make_async_copy(x_hbm.at[src_b[r]], rows_b[b], gsem[b]).wait()

    def scatter_start(sb, r):
        pltpu.async_copy(srows[sb], acc_sh.at[dst_b[r].at[0]],
                         ssem[sb], add=True)

    def scatter_wait(sb, r):
        pltpu.make_async_copy(srows[sb], acc_sh.at[dst_b[r].at[0]],
                              ssem[sb]).wait()

    idx_start(0, 0)
    idx_start(1, 1)
    idx_wait(0, 0)
    gather_start(0, 0)
    plsc.subcore_barrier()

    def round_body(rr, _):
        for u in range(UNROLL):
            i = rr * UNROLL + u
            b = sb = u % NBUF
            r = u % NIDX

            @pl.when((i >= 2) & (i - 2 < MAIN))
            def _():
                scatter_wait(sb, (u + 2) % NIDX)

            @pl.when(i + 2 < MAIN)
            def _():
                idx_start(i + 2, (u + 2) % NIDX)

            @pl.when(i + 1 < MAIN)
            def _():
                idx_wait(i + 1, (u + 1) % NIDX)
                gather_start((u + 1) % NBUF, (u + 1) % NIDX)

            @pl.when(i < MAIN)
            def _():
                gather_wait(b, r)
                _scale_rows(rows_b[b], srows[sb], w_b[r])
                scatter_start(sb, r)
        return 0
    lax.fori_loop(0, ROUNDS, round_body, 0)

    for k in range(max(MAIN - 2, ROUNDS * UNROLL - 2), MAIN):
        scatter_wait(k % NBUF, k % NIDX)

    plsc.subcore_barrier()

    @pl.when(s < NS - 1)
    def _():
        pltpu.sync_copy(acc_sh.at[pl.ds(s * STRIPE, STRIPE)],
                        out_hbm.at[c, pl.ds(s * STRIPE, STRIPE)])

    @pl.when(s == NS - 1)
    def _():
        pltpu.sync_copy(acc_sh.at[pl.ds((NS - 1) * STRIPE, LAST)],
                        out_hbm.at[c, pl.ds((NS - 1) * STRIPE, LAST)])


@functools.lru_cache(maxsize=None)
def _spmm_sc():
    return pl.kernel(
        _spmm_sc_body,
        out_type=jax.ShapeDtypeStruct((NC, N, D), jnp.float32),
        mesh=plsc.VectorSubcoreMesh(core_axis_name="c", subcore_axis_name="s",
                                    num_cores=NC, num_subcores=NS),
        scratch_types=[
            [pltpu.VMEM((CHUNK,), jnp.int32) for _ in range(NIDX)],
            [pltpu.VMEM((CHUNK, D), jnp.float32) for _ in range(NBUF)],
            [pltpu.VMEM((CHUNK,), jnp.float32) for _ in range(NIDX)],
            [pltpu.VMEM((1, CHUNK), jnp.int32) for _ in range(NIDX)],
            [pltpu.VMEM((CHUNK, D), jnp.float32) for _ in range(NBUF)],
            [pltpu.SemaphoreType.DMA for _ in range(NBUF)],
            [pltpu.SemaphoreType.DMA for _ in range(NIDX)],
            [pltpu.SemaphoreType.DMA for _ in range(NBUF)],
            pltpu.VMEM_SHARED((N, D), jnp.float32),
        ],
    )


def _pad_adj(src, dst, w):
    zi = jnp.zeros((EPAD,), jnp.int32)
    return (jnp.concatenate([src, zi]), jnp.concatenate([dst, zi]),
            jnp.concatenate([w, jnp.zeros((EPAD,), jnp.float32)]))


def _spmm(x, adj):
    src, dst, w = adj
    return _spmm_sc()(x, src, dst, w, jnp.zeros((N, D), jnp.float32))


BN = 2000


def _mm_body(x_ref, w_ref, b_ref, o_ref):
    o_ref[...] = jnp.dot(x_ref[...], w_ref[...],
                         preferred_element_type=jnp.float32) + b_ref[...]


def _mm_fused_body(p_ref, w_ref, b_ref, o_ref):
    x = jax.nn.relu(p_ref[0] + p_ref[1])
    o_ref[...] = jnp.dot(x, w_ref[...],
                         preferred_element_type=jnp.float32) + b_ref[...]


def _mm_fused_id_body(p_ref, h_ref, w_ref, b_ref, o_ref):
    x = jax.nn.relu(p_ref[0] + p_ref[1] + h_ref[...])
    o_ref[...] = jnp.dot(x, w_ref[...],
                         preferred_element_type=jnp.float32) + b_ref[...]


def _relu_sum_body(p_ref, o_ref):
    o_ref[...] = jax.nn.relu(p_ref[0] + p_ref[1])


def _norm_body(p_ref, o_ref):
    r = p_ref[0] + p_ref[1]
    n = jnp.sqrt(jnp.sum(r * r, axis=-1, keepdims=True))
    o_ref[...] = r / (n + 1e-9)


_row_spec = pl.BlockSpec((BN, D), lambda i: (i, 0))
_p_spec = pl.BlockSpec((2, BN, D), lambda i: (0, i, 0))
_w_spec = pl.BlockSpec((D, D), lambda i: (0, 0))
_b_spec = pl.BlockSpec((1, D), lambda i: (0, 0))
_out_nd = jax.ShapeDtypeStruct((N, D), jnp.float32)


def _mm(x, w, b):
    return pl.pallas_call(
        _mm_body, grid=(N // BN,),
        in_specs=[_row_spec, _w_spec, _b_spec],
        out_specs=_row_spec, out_shape=_out_nd)(x, w, b.reshape(1, D))


def _mm_fused(p, w, b):
    return pl.pallas_call(
        _mm_fused_body, grid=(N // BN,),
        in_specs=[_p_spec, _w_spec, _b_spec],
        out_specs=_row_spec, out_shape=_out_nd)(p, w, b.reshape(1, D))


def _mm_fused_id(p, h, w, b):
    return pl.pallas_call(
        _mm_fused_id_body, grid=(N // BN,),
        in_specs=[_p_spec, _row_spec, _w_spec, _b_spec],
        out_specs=_row_spec, out_shape=_out_nd)(p, h, w, b.reshape(1, D))


def _relu_sum(p):
    return pl.pallas_call(
        _relu_sum_body, grid=(N // BN,),
        in_specs=[_p_spec], out_specs=_row_spec, out_shape=_out_nd)(p)


def _norm(p):
    return pl.pallas_call(
        _norm_body, grid=(N // BN,),
        in_specs=[_p_spec], out_specs=_row_spec, out_shape=_out_nd)(p)


def _one_type(feat, W1, b1, W2, b2, src, dst, w, src0, dst0, w0, identity):
    adj = _pad_adj(src, dst, w)
    adj0 = _pad_adj(src0, dst0, w0)
    h = _mm(feat, W1, b1)
    p1 = _spmm(h, adj)
    if identity:
        h2in = _mm_fused_id(p1, h, W2, b2)
    else:
        h2in = _mm_fused(p1, W2, b2)
    p2 = _spmm(h2in, adj)
    x2 = _relu_sum(p2)
    p3 = _spmm(x2, adj0)
    return _norm(p3)


def kernel(feat_1, W1_1, b1_1, W2_1, b2_1, src_11, dst_11, w_11,
           src_01, dst_01, w_01,
           feat_2, W1_2, b1_2, W2_2, b2_2, src_22, dst_22, w_22,
           src_02, dst_02, w_02,
           feat_3, W1_3, b1_3, W2_3, b2_3, src_33, dst_33, w_33,
           src_03, dst_03, w_03, epoch):
    r1 = _one_type(feat_1, W1_1, b1_1, W2_1, b2_1, src_11, dst_11, w_11,
                   src_01, dst_01, w_01, identity=True)
    r2 = _one_type(feat_2, W1_2, b1_2, W2_2, b2_2, src_22, dst_22, w_22,
                   src_02, dst_02, w_02, identity=False)
    r3 = _one_type(feat_3, W1_3, b1_3, W2_3, b2_3, src_33, dst_33, w_33,
                   src_03, dst_03, w_03, identity=False)
    return jnp.stack([r1, r2, r3], axis=0)
